# baseline (speedup 1.0000x reference)
import dataclasses
import functools

import jax
import jax.numpy as jnp
from jax import lax
from jax.experimental import pallas as pl
from jax.experimental.pallas import tpu as pltpu
from jax.experimental.pallas import tpu_sc as plsc

N = 10000
D = 128
E = 320000
L = 320000
R = E // 128
RP = 2528
PW = RP // 32
NC = 2
NS = 16
HPAD = 10240
NPAD = 10240


def _sc_mesh():
    return plsc.VectorSubcoreMesh(core_axis_name="c", subcore_axis_name="s")


def _sc_params():
    cp = pltpu.CompilerParams()
    fields = pltpu.CompilerParams.__dataclass_fields__
    if "needs_layout_passes" in fields:
        cp = dataclasses.replace(cp, needs_layout_passes=False)
    if "use_tc_tiling_on_sc" in fields:
        cp = dataclasses.replace(cp, use_tc_tiling_on_sc=False)
    return cp


def _deg_hist(dst3d, zeros_h):
    @functools.partial(
        pl.kernel,
        out_type=jax.ShapeDtypeStruct((NC, 1, HPAD), jnp.float32),
        mesh=_sc_mesh(),
        scratch_types=[
            pltpu.VMEM((PW, 1, 128), jnp.int32),
            pltpu.VMEM((128,), jnp.float32),
            pltpu.VMEM_SHARED((HPAD,), jnp.float32),
        ],
    )
    def k(dst_hbm, z_hbm, out_hbm, idx_v, ones_v, acc):
        c = lax.axis_index("c")
        s = lax.axis_index("s")
        for j in range(8):
            ones_v[pl.ds(16 * j, 16)] = jnp.full((16,), 1.0, jnp.float32)
        seg = HPAD // NS
        pltpu.sync_copy(z_hbm.at[pl.ds(s * seg, seg)], acc.at[pl.ds(s * seg, seg)])
        half = R // NC
        r0 = half * c + (half * s) // NS
        r1 = half * c + (half * (s + 1)) // NS
        n = r1 - r0
        pltpu.sync_copy(dst_hbm.at[pl.ds(r0, PW)], idx_v)
        plsc.subcore_barrier()

        @pl.loop(0, n)
        def _(i):
            pltpu.sync_copy(ones_v, acc.at[idx_v.at[i, 0]], add=True)

        plsc.subcore_barrier()
        pltpu.sync_copy(acc.at[pl.ds(s * seg, seg)],
                        out_hbm.at[c, 0, pl.ds(s * seg, seg)])

    return k(dst3d, zeros_h)


def _aggregate(y, src3d, dst3d, zeros2d):
    @functools.partial(
        pl.kernel,
        out_type=jax.ShapeDtypeStruct((NC, NPAD, D), jnp.float32),
        mesh=_sc_mesh(),
        scratch_types=[
            pltpu.VMEM((PW, 1, 128), jnp.int32),
            pltpu.VMEM((1, 128), jnp.int32),
            pltpu.VMEM((1, 128), jnp.int32),
            pltpu.VMEM((128, D), jnp.float32),
            pltpu.VMEM((128, D), jnp.float32),
            pltpu.VMEM_SHARED((NPAD, D), jnp.float32),
            pltpu.SemaphoreType.DMA,
            pltpu.SemaphoreType.DMA,
        ],
    )
    def k(y_hbm, s_hbm, d_hbm, z_hbm, out_hbm, sv, dv0, dv1, val0, val1, acc,
          sem0, sem1):
        c = lax.axis_index("c")
        s = lax.axis_index("s")
        rows = NPAD // NS
        pltpu.sync_copy(z_hbm.at[pl.ds(s * rows, rows)],
                        acc.at[pl.ds(s * rows, rows)])
        half = R // NC
        r0 = half * c + (half * s) // NS
        r1 = half * c + (half * (s + 1)) // NS
        n = r1 - r0
        pltpu.sync_copy(s_hbm.at[pl.ds(r0, PW)], sv)
        plsc.subcore_barrier()

        def gat(i, val, dv, sem):
            pltpu.async_copy(d_hbm.at[r0 + i], dv, sem)
            pltpu.async_copy(y_hbm.at[sv.at[i, 0]], val, sem)

        def wait_gat(val, dv, sem):
            pltpu.make_async_copy(d_hbm.at[0], dv, sem).wait()
            pltpu.make_async_copy(y_hbm.at[sv.at[0, 0]], val, sem).wait()

        @pl.when(n >= 1)
        def _():
            gat(0, val0, dv0, sem0)

        @pl.when(n >= 2)
        def _():
            gat(1, val1, dv1, sem1)

        @pl.loop(0, n, step=2)
        def _(i):
            wait_gat(val0, dv0, sem0)
            pltpu.sync_copy(val0, acc.at[dv0.at[0]], add=True)

            @pl.when(i + 2 < n)
            def _():
                gat(i + 2, val0, dv0, sem0)

            @pl.when(i + 1 < n)
            def _():
                wait_gat(val1, dv1, sem1)
                pltpu.sync_copy(val1, acc.at[dv1.at[0]], add=True)

                @pl.when(i + 3 < n)
                def _():
                    gat(i + 3, val1, dv1, sem1)

        plsc.subcore_barrier()
        pltpu.sync_copy(acc.at[pl.ds(s * rows, rows)],
                        out_hbm.at[c].at[pl.ds(s * rows, rows)])

    return k(y, src3d, dst3d, zeros2d)


def _pair_decode(g, gB, a3d, b3d):
    @functools.partial(
        pl.kernel,
        out_type=jax.ShapeDtypeStruct((L, D // 2), jnp.float32),
        mesh=_sc_mesh(),
        compiler_params=_sc_params(),
        scratch_types=[
            pltpu.VMEM((PW, 1, 128), jnp.int32),
            pltpu.VMEM((PW, 1, 128), jnp.int32),
            pltpu.VMEM((128, D // 2), jnp.float32),
            pltpu.VMEM((128, D // 2), jnp.float32),
            pltpu.VMEM((128, D // 2), jnp.float32),
            pltpu.VMEM((128, D // 2), jnp.float32),
            pltpu.VMEM((128, D // 2), jnp.float32),
            pltpu.VMEM((128, D // 2), jnp.float32),
            pltpu.SemaphoreType.DMA,
            pltpu.SemaphoreType.DMA,
            pltpu.SemaphoreType.DMA,
            pltpu.SemaphoreType.DMA,
        ],
    )
    def k(g_hbm, gb_hbm, a_hbm, b_hbm, out_hbm, av, bv,
          va0, vb0, ob0, va1, vb1, ob1, semg0, semg1, sems0, sems1):
        c = lax.axis_index("c")
        s = lax.axis_index("s")
        w = s * NC + c
        r0 = (R * w) // (NC * NS)
        r1 = (R * (w + 1)) // (NC * NS)
        n = r1 - r0
        pltpu.sync_copy(a_hbm.at[pl.ds(r0, PW)], av)
        pltpu.sync_copy(b_hbm.at[pl.ds(r0, PW)], bv)

        def gat(i, va, vb, sem):
            pltpu.async_copy(g_hbm.at[av.at[i, 0]], va, sem)
            pltpu.async_copy(gb_hbm.at[bv.at[i, 0]], vb, sem)

        def wait_gat(va, vb, sem):
            pltpu.make_async_copy(g_hbm.at[av.at[0, 0]], va, sem).wait()
            pltpu.make_async_copy(g_hbm.at[av.at[0, 0]], vb, sem).wait()

        def wait_store(ob, sem):
            pltpu.make_async_copy(ob, out_hbm.at[pl.ds(0, 128)], sem).wait()

        zero = jnp.zeros((32,), jnp.bfloat16)

        def compute(va, vb, ob):
            @pl.loop(0, 128)
            def _(i):
                for j in range(4):
                    sl = pl.ds(16 * j, 16)
                    a16 = plsc.bitcast(va[i, sl], jnp.bfloat16)
                    b16 = plsc.bitcast(vb[i, sl], jnp.bfloat16)
                    r16 = jnp.maximum(b16 - a16, zero)
                    ob[i, sl] = plsc.bitcast(r16, jnp.float32)

        @pl.when(n >= 1)
        def _():
            gat(0, va0, vb0, semg0)

        @pl.when(n >= 2)
        def _():
            gat(1, va1, vb1, semg1)

        @pl.loop(0, n, step=2)
        def _(i):
            wait_gat(va0, vb0, semg0)

            @pl.when(i >= 2)
            def _():
                wait_store(ob0, sems0)

            compute(va0, vb0, ob0)
            pltpu.async_copy(ob0, out_hbm.at[pl.ds((r0 + i) * 128, 128)], sems0)

            @pl.when(i + 2 < n)
            def _():
                gat(i + 2, va0, vb0, semg0)

            @pl.when(i + 1 < n)
            def _():
                wait_gat(va1, vb1, semg1)

                @pl.when(i >= 2)
                def _():
                    wait_store(ob1, sems1)

                compute(va1, vb1, ob1)
                pltpu.async_copy(
                    ob1, out_hbm.at[pl.ds((r0 + i + 1) * 128, 128)], sems1)

                @pl.when(i + 3 < n)
                def _():
                    gat(i + 3, va1, vb1, semg1)

        @pl.when(n >= 1)
        def _():
            wait_store(ob0, sems0)

        @pl.when(n >= 2)
        def _():
            wait_store(ob1, sems1)

    return k(g, gB, a3d, b3d)


def _tc_prep(x, W_gcn, hist3d):
    def body(x_ref, w_ref, h_ref, y_ref, st_ref, dis_ref):
        xw = jnp.dot(x_ref[...], w_ref[...],
                     preferred_element_type=jnp.float32)
        hv = h_ref[...]
        deg = hv[0] + hv[1] + 1.0
        dis = lax.rsqrt(deg)[:N]
        ideg = (1.0 / deg)[:N]
        y_ref[...] = xw * dis
        st_ref[...] = xw * ideg
        dis_ref[...] = dis

    return pl.pallas_call(
        body,
        out_shape=[
            jax.ShapeDtypeStruct((N, D), jnp.float32),
            jax.ShapeDtypeStruct((N, D), jnp.float32),
            jax.ShapeDtypeStruct((N, 1), jnp.float32),
        ],
    )(x, W_gcn, hist3d)


def _bf16_hi_bits(x):
    u = lax.bitcast_convert_type(x, jnp.uint32)
    rounded = u + jnp.uint32(0x7FFF) + ((u >> 16) & jnp.uint32(1))
    return rounded & jnp.uint32(0xFFFF0000)


def _pack_pair(lo, hi):
    w = (_bf16_hi_bits(lo) >> 16) | _bf16_hi_bits(hi)
    return lax.bitcast_convert_type(w, jnp.float32)


def _tc_gtable(p, st, dis_col, W1e, W1o, b_gcn2d, b1e, b1o):
    def body(p_ref, st_ref, dis_ref, w1e_ref, w1o_ref, bg_ref, b1e_ref,
             b1o_ref, g_ref, gb_ref):
        pv = p_ref[...]
        h = pv[0, :N] + pv[1, :N]
        h = h * dis_ref[...] + st_ref[...] + bg_ref[...]
        ge = jnp.dot(h, w1e_ref[...], preferred_element_type=jnp.float32)
        go = jnp.dot(h, w1o_ref[...], preferred_element_type=jnp.float32)
        g_ref[...] = _pack_pair(ge, go)
        gb_ref[...] = _pack_pair(ge + b1e_ref[...], go + b1o_ref[...])

    return pl.pallas_call(
        body,
        out_shape=[
            jax.ShapeDtypeStruct((N, D // 2), jnp.float32),
            jax.ShapeDtypeStruct((N, D // 2), jnp.float32),
        ],
    )(p, st, dis_col, W1e, W1o, b_gcn2d, b1e, b1o)


def _tc_mlp(z1, W2e, W2o, b2_2d, W3, b3_2d):
    blk = 3200
    grid = L // blk

    def body(z_ref, w2e_ref, w2o_ref, b2_ref, w3_ref, b3_ref, o_ref):
        u = lax.bitcast_convert_type(z_ref[...], jnp.uint32)
        lo = lax.bitcast_convert_type(u << 16, jnp.float32)
        hi = lax.bitcast_convert_type(u & jnp.uint32(0xFFFF0000), jnp.float32)
        h1 = jnp.maximum(
            jnp.dot(lo, w2e_ref[...], preferred_element_type=jnp.float32)
            + jnp.dot(hi, w2o_ref[...], preferred_element_type=jnp.float32)
            + b2_ref[...], 0.0)
        o_ref[...] = (
            jnp.dot(h1, w3_ref[...], preferred_element_type=jnp.float32)
            + b3_ref[...])

    return pl.pallas_call(
        body,
        grid=(grid,),
        in_specs=[
            pl.BlockSpec((blk, D // 2), lambda i: (i, 0)),
            pl.BlockSpec((D // 2, 32), lambda i: (0, 0)),
            pl.BlockSpec((D // 2, 32), lambda i: (0, 0)),
            pl.BlockSpec((1, 32), lambda i: (0, 0)),
            pl.BlockSpec((32, 1), lambda i: (0, 0)),
            pl.BlockSpec((1, 1), lambda i: (0, 0)),
        ],
        out_specs=pl.BlockSpec((blk, 1), lambda i: (i, 0)),
        out_shape=jax.ShapeDtypeStruct((L, 1), jnp.float32),
    )(z1, W2e, W2o, b2_2d, W3, b3_2d)


def _pad3d(v):
    return jnp.pad(v, (0, RP * 128 - v.shape[0])).reshape(RP, 1, 128)


def kernel(x, edge_index, edge_label_index, W_gcn, b_gcn, W1, b1, W2, b2, W3, b3):
    ei = edge_index.astype(jnp.int32)
    eli = edge_label_index.astype(jnp.int32)
    src3d = _pad3d(ei[0])
    dst3d = _pad3d(ei[1])
    a3d = _pad3d(eli[0])
    b3d = _pad3d(eli[1])

    zeros_h = jnp.zeros((HPAD,), jnp.float32)
    zeros2d = jnp.zeros((NPAD, D), jnp.float32)

    hist = _deg_hist(dst3d, zeros_h)
    y, st, dis_col = _tc_prep(x, W_gcn, hist.reshape(NC, HPAD, 1))
    p = _aggregate(y, src3d, dst3d, zeros2d)
    g, gB = _tc_gtable(p, st, dis_col, W1[:, 0::2], W1[:, 1::2],
                       b_gcn.reshape(1, D),
                       b1[0::2].reshape(1, D // 2), b1[1::2].reshape(1, D // 2))
    z1 = _pair_decode(g, gB, a3d, b3d)
    prob = _tc_mlp(z1, W2[0::2], W2[1::2], b2.reshape(1, 32), W3,
                   b3.reshape(1, 1))
    return prob.reshape(-1)


# speedup vs baseline: 1.1078x; 1.1078x over previous
import dataclasses
import functools

import jax
import jax.numpy as jnp
from jax import lax
from jax.experimental import pallas as pl
from jax.experimental.pallas import tpu as pltpu
from jax.experimental.pallas import tpu_sc as plsc

N = 10000
D = 128
E = 320000
L = 320000
R = E // 128
RP = 2528
PW = RP // 32
NC = 2
NS = 16
HPAD = 10240
NPAD = 10240


def _sc_mesh():
    return plsc.VectorSubcoreMesh(core_axis_name="c", subcore_axis_name="s")


def _sc_params():
    cp = pltpu.CompilerParams()
    fields = pltpu.CompilerParams.__dataclass_fields__
    if "needs_layout_passes" in fields:
        cp = dataclasses.replace(cp, needs_layout_passes=False)
    if "use_tc_tiling_on_sc" in fields:
        cp = dataclasses.replace(cp, use_tc_tiling_on_sc=False)
    return cp


def _deg_hist(dst3d, zeros_h):
    @functools.partial(
        pl.kernel,
        out_type=jax.ShapeDtypeStruct((NC, 1, HPAD), jnp.float32),
        mesh=_sc_mesh(),
        scratch_types=[
            pltpu.VMEM((PW, 1, 128), jnp.int32),
            pltpu.VMEM((128,), jnp.float32),
            pltpu.VMEM_SHARED((HPAD,), jnp.float32),
        ],
    )
    def k(dst_hbm, z_hbm, out_hbm, idx_v, ones_v, acc):
        c = lax.axis_index("c")
        s = lax.axis_index("s")
        for j in range(8):
            ones_v[pl.ds(16 * j, 16)] = jnp.full((16,), 1.0, jnp.float32)
        seg = HPAD // NS
        pltpu.sync_copy(z_hbm.at[pl.ds(s * seg, seg)], acc.at[pl.ds(s * seg, seg)])
        half = R // NC
        r0 = half * c + (half * s) // NS
        r1 = half * c + (half * (s + 1)) // NS
        n = r1 - r0
        pltpu.sync_copy(dst_hbm.at[pl.ds(r0, PW)], idx_v)
        plsc.subcore_barrier()

        @pl.loop(0, n)
        def _(i):
            pltpu.sync_copy(ones_v, acc.at[idx_v.at[i, 0]], add=True)

        plsc.subcore_barrier()
        pltpu.sync_copy(acc.at[pl.ds(s * seg, seg)],
                        out_hbm.at[c, 0, pl.ds(s * seg, seg)])

    return k(dst3d, zeros_h)


def _aggregate(y, src3d, dst3d, zeros2d):
    @functools.partial(
        pl.kernel,
        out_type=jax.ShapeDtypeStruct((NC, NPAD, D), jnp.float32),
        mesh=_sc_mesh(),
        scratch_types=[
            pltpu.VMEM((PW, 1, 128), jnp.int32),
            pltpu.VMEM((1, 128), jnp.int32),
            pltpu.VMEM((1, 128), jnp.int32),
            pltpu.VMEM((128, D), jnp.float32),
            pltpu.VMEM((128, D), jnp.float32),
            pltpu.VMEM_SHARED((NPAD, D), jnp.float32),
            pltpu.SemaphoreType.DMA,
            pltpu.SemaphoreType.DMA,
        ],
    )
    def k(y_hbm, s_hbm, d_hbm, z_hbm, out_hbm, sv, dv0, dv1, val0, val1, acc,
          sem0, sem1):
        c = lax.axis_index("c")
        s = lax.axis_index("s")
        rows = NPAD // NS
        pltpu.sync_copy(z_hbm.at[pl.ds(s * rows, rows)],
                        acc.at[pl.ds(s * rows, rows)])
        half = R // NC
        r0 = half * c + (half * s) // NS
        r1 = half * c + (half * (s + 1)) // NS
        n = r1 - r0
        pltpu.sync_copy(s_hbm.at[pl.ds(r0, PW)], sv)
        plsc.subcore_barrier()

        def gat(i, val, dv, sem):
            pltpu.async_copy(d_hbm.at[r0 + i], dv, sem)
            pltpu.async_copy(y_hbm.at[sv.at[i, 0]], val, sem)

        def wait_gat(val, dv, sem):
            pltpu.make_async_copy(d_hbm.at[0], dv, sem).wait()
            pltpu.make_async_copy(y_hbm.at[sv.at[0, 0]], val, sem).wait()

        @pl.when(n >= 1)
        def _():
            gat(0, val0, dv0, sem0)

        @pl.when(n >= 2)
        def _():
            gat(1, val1, dv1, sem1)

        @pl.loop(0, n, step=2)
        def _(i):
            wait_gat(val0, dv0, sem0)
            pltpu.sync_copy(val0, acc.at[dv0.at[0]], add=True)

            @pl.when(i + 2 < n)
            def _():
                gat(i + 2, val0, dv0, sem0)

            @pl.when(i + 1 < n)
            def _():
                wait_gat(val1, dv1, sem1)
                pltpu.sync_copy(val1, acc.at[dv1.at[0]], add=True)

                @pl.when(i + 3 < n)
                def _():
                    gat(i + 3, val1, dv1, sem1)

        plsc.subcore_barrier()
        pltpu.sync_copy(acc.at[pl.ds(s * rows, rows)],
                        out_hbm.at[c].at[pl.ds(s * rows, rows)])

    return k(y, src3d, dst3d, zeros2d)


def _pair_decode(g, gB, a3d, b3d):
    @functools.partial(
        pl.kernel,
        out_type=jax.ShapeDtypeStruct((L, D), jnp.float32),
        mesh=_sc_mesh(),
        scratch_types=[
            pltpu.VMEM((PW, 1, 128), jnp.int32),
            pltpu.VMEM((PW, 1, 128), jnp.int32),
            pltpu.VMEM((128, D), jnp.float32),
            pltpu.VMEM((128, D), jnp.float32),
            pltpu.VMEM((128, D), jnp.float32),
            pltpu.VMEM((128, D), jnp.float32),
            pltpu.VMEM((128, D), jnp.float32),
            pltpu.VMEM((128, D), jnp.float32),
            pltpu.SemaphoreType.DMA,
            pltpu.SemaphoreType.DMA,
            pltpu.SemaphoreType.DMA,
            pltpu.SemaphoreType.DMA,
        ],
    )
    def k(g_hbm, gb_hbm, a_hbm, b_hbm, out_hbm, av, bv,
          va0, vb0, ob0, va1, vb1, ob1, semg0, semg1, sems0, sems1):
        c = lax.axis_index("c")
        s = lax.axis_index("s")
        w = s * NC + c
        r0 = (R * w) // (NC * NS)
        r1 = (R * (w + 1)) // (NC * NS)
        n = r1 - r0
        pltpu.sync_copy(a_hbm.at[pl.ds(r0, PW)], av)
        pltpu.sync_copy(b_hbm.at[pl.ds(r0, PW)], bv)

        def gat(i, va, vb, sem):
            pltpu.async_copy(g_hbm.at[av.at[i, 0]], va, sem)
            pltpu.async_copy(gb_hbm.at[bv.at[i, 0]], vb, sem)

        def wait_gat(va, vb, sem):
            pltpu.make_async_copy(g_hbm.at[av.at[0, 0]], va, sem).wait()
            pltpu.make_async_copy(g_hbm.at[av.at[0, 0]], vb, sem).wait()

        def wait_store(ob, sem):
            pltpu.make_async_copy(ob, out_hbm.at[pl.ds(0, 128)], sem).wait()

        def compute(va, vb, ob):
            @pl.loop(0, 128)
            def _(i):
                for j in range(8):
                    sl = pl.ds(16 * j, 16)
                    ob[i, sl] = jnp.maximum(vb[i, sl] - va[i, sl], 0.0)

        @pl.when(n >= 1)
        def _():
            gat(0, va0, vb0, semg0)

        @pl.when(n >= 2)
        def _():
            gat(1, va1, vb1, semg1)

        @pl.loop(0, n, step=2)
        def _(i):
            wait_gat(va0, vb0, semg0)

            @pl.when(i >= 2)
            def _():
                wait_store(ob0, sems0)

            compute(va0, vb0, ob0)
            pltpu.async_copy(ob0, out_hbm.at[pl.ds((r0 + i) * 128, 128)], sems0)

            @pl.when(i + 2 < n)
            def _():
                gat(i + 2, va0, vb0, semg0)

            @pl.when(i + 1 < n)
            def _():
                wait_gat(va1, vb1, semg1)

                @pl.when(i >= 2)
                def _():
                    wait_store(ob1, sems1)

                compute(va1, vb1, ob1)
                pltpu.async_copy(
                    ob1, out_hbm.at[pl.ds((r0 + i + 1) * 128, 128)], sems1)

                @pl.when(i + 3 < n)
                def _():
                    gat(i + 3, va1, vb1, semg1)

        @pl.when(n >= 1)
        def _():
            wait_store(ob0, sems0)

        @pl.when(n >= 2)
        def _():
            wait_store(ob1, sems1)

    return k(g, gB, a3d, b3d)


def _tc_prep(x, W_gcn, hist3d):
    def body(x_ref, w_ref, h_ref, y_ref, st_ref, dis_ref):
        xw = jnp.dot(x_ref[...], w_ref[...],
                     preferred_element_type=jnp.float32)
        hv = h_ref[...]
        deg = hv[0] + hv[1] + 1.0
        dis = lax.rsqrt(deg)[:N]
        ideg = (1.0 / deg)[:N]
        y_ref[...] = xw * dis
        st_ref[...] = xw * ideg
        dis_ref[...] = dis

    return pl.pallas_call(
        body,
        out_shape=[
            jax.ShapeDtypeStruct((N, D), jnp.float32),
            jax.ShapeDtypeStruct((N, D), jnp.float32),
            jax.ShapeDtypeStruct((N, 1), jnp.float32),
        ],
    )(x, W_gcn, hist3d)


def _tc_gtable(p, st, dis_col, W1, b_gcn2d, b1_2d):
    def body(p_ref, st_ref, dis_ref, w1_ref, bg_ref, b1_ref, g_ref, gb_ref):
        pv = p_ref[...]
        h = pv[0, :N] + pv[1, :N]
        h = h * dis_ref[...] + st_ref[...] + bg_ref[...]
        g = jnp.dot(h, w1_ref[...], preferred_element_type=jnp.float32)
        g_ref[...] = g
        gb_ref[...] = g + b1_ref[...]

    return pl.pallas_call(
        body,
        out_shape=[
            jax.ShapeDtypeStruct((N, D), jnp.float32),
            jax.ShapeDtypeStruct((N, D), jnp.float32),
        ],
    )(p, st, dis_col, W1, b_gcn2d, b1_2d)


def _tc_mlp(z1, W2, b2_2d, W3, b3_2d):
    blk = 3200
    grid = L // blk

    def body(z_ref, w2_ref, b2_ref, w3_ref, b3_ref, o_ref):
        z = z_ref[...]
        h1 = jnp.maximum(
            jnp.dot(z, w2_ref[...], preferred_element_type=jnp.float32)
            + b2_ref[...], 0.0)
        o_ref[...] = (
            jnp.dot(h1, w3_ref[...], preferred_element_type=jnp.float32)
            + b3_ref[...])

    return pl.pallas_call(
        body,
        grid=(grid,),
        in_specs=[
            pl.BlockSpec((blk, D), lambda i: (i, 0)),
            pl.BlockSpec((D, 32), lambda i: (0, 0)),
            pl.BlockSpec((1, 32), lambda i: (0, 0)),
            pl.BlockSpec((32, 1), lambda i: (0, 0)),
            pl.BlockSpec((1, 1), lambda i: (0, 0)),
        ],
        out_specs=pl.BlockSpec((blk, 1), lambda i: (i, 0)),
        out_shape=jax.ShapeDtypeStruct((L, 1), jnp.float32),
    )(z1, W2, b2_2d, W3, b3_2d)


def _pad3d(v):
    return jnp.pad(v, (0, RP * 128 - v.shape[0])).reshape(RP, 1, 128)


def kernel(x, edge_index, edge_label_index, W_gcn, b_gcn, W1, b1, W2, b2, W3, b3):
    ei = edge_index.astype(jnp.int32)
    eli = edge_label_index.astype(jnp.int32)
    src3d = _pad3d(ei[0])
    dst3d = _pad3d(ei[1])
    a3d = _pad3d(eli[0])
    b3d = _pad3d(eli[1])

    zeros_h = jnp.zeros((HPAD,), jnp.float32)
    zeros2d = jnp.zeros((NPAD, D), jnp.float32)

    hist = _deg_hist(dst3d, zeros_h)
    y, st, dis_col = _tc_prep(x, W_gcn, hist.reshape(NC, HPAD, 1))
    p = _aggregate(y, src3d, dst3d, zeros2d)
    g, gB = _tc_gtable(p, st, dis_col, W1,
                       b_gcn.reshape(1, D), b1.reshape(1, D))
    z1 = _pair_decode(g, gB, a3d, b3d)
    prob = _tc_mlp(z1, W2, b2.reshape(1, 32), W3, b3.reshape(1, 1))
    return prob.reshape(-1)
